# Initial kernel scaffold; baseline (speedup 1.0000x reference)
import jax
import jax.numpy as jnp
from jax.experimental import pallas as pl


def kernel(masks, poking_locations, anchor_boxes):
    raise NotImplementedError("write your pallas kernel here")


# speedup vs baseline: 2.3116x; 2.3116x over previous
import functools

import jax
import jax.numpy as jnp
from jax import lax
from jax.experimental import pallas as pl
from jax.experimental.pallas import tpu as pltpu
from jax.experimental.pallas import tpu_sc as plsc

_NC = 2
_NS = 16
_NW = _NC * _NS

_M = 16
_A = 9
_G = 32
_H = 512
_L = 16

_N_MASK_TASKS = _M * _A
_N_TASKS = _N_MASK_TASKS + _A
_SLOTS = -(-_N_TASKS // _NW)


def _sc_body(masks_hbm, poking_hbm, ridx_hbm, ys_hbm, areas_hbm, wbc_hbm,
             ious_hbm, poke_hbm,
             ys_v, areas_v, wbc_v, idx_v, rows_v, out_v, sem):
    wid = lax.axis_index("s") * _NC + lax.axis_index("c")

    pltpu.sync_copy(ys_hbm, ys_v)
    pltpu.sync_copy(areas_hbm, areas_v)
    pltpu.sync_copy(wbc_hbm, wbc_v)

    def compute_block(a, area_vec, use_iou):
        def row_body(i, carry):
            ri = jnp.full((_L,), i, jnp.int32)
            ri2 = ri + _G
            if use_iou:
                w_vec = wbc_v[a, i, pl.ds(0, _L)]
            for jc in range(2):
                cy1 = ys_v[a, pl.ds(jc * _L, _L)]
                cy3 = ys_v[a, pl.ds(_G + jc * _L, _L)]
                v01 = plsc.load_gather(rows_v, [ri, cy1])
                v03 = plsc.load_gather(rows_v, [ri, cy3])
                v21 = plsc.load_gather(rows_v, [ri2, cy1])
                v23 = plsc.load_gather(rows_v, [ri2, cy3])
                inter = v01 + v23 - v03 - v21
                if use_iou:
                    box = w_vec * (cy3 - cy1).astype(jnp.float32)
                    res = inter / jnp.maximum(area_vec + box - inter, 1.0)
                else:
                    res = inter
                out_v[i, pl.ds(jc * _L, _L)] = res
            return carry

        lax.fori_loop(0, _G, row_body, 0)

    for k in range(_SLOTS):
        t = wid + _NW * k

        @pl.when(t < _N_MASK_TASKS)
        def _mask_task():
            p = lax.div(t, _A)
            a = lax.rem(t, _A)
            pltpu.sync_copy(ridx_hbm.at[t], idx_v)
            pltpu.async_copy(masks_hbm.at[idx_v], rows_v, sem).wait()
            area_vec = areas_v[p, pl.ds(0, _L)]
            compute_block(a, area_vec, True)
            pltpu.sync_copy(out_v, ious_hbm.at[t])

        @pl.when(jnp.logical_and(t >= _N_MASK_TASKS, t < _N_TASKS))
        def _poke_task():
            a = t - _N_MASK_TASKS
            pltpu.sync_copy(ridx_hbm.at[t], idx_v)
            pltpu.async_copy(poking_hbm.at[idx_v], rows_v, sem).wait()
            compute_block(a, None, False)
            pltpu.sync_copy(out_v, poke_hbm.at[a])


@jax.jit
def kernel(masks, poking_locations, anchor_boxes):
    masks2d = masks.reshape(_M * _H, _H)
    poking2d = poking_locations.reshape(_H, _H)

    ab = anchor_boxes[0]
    x0 = ab[:, :, 0, 0]
    x2 = ab[:, :, 0, 2]
    y1 = ab[:, 0, :, 1]
    y3 = ab[:, 0, :, 3]
    xs = jnp.concatenate([x0, x2], axis=1).astype(jnp.int32)
    ys = jnp.concatenate([y1, y3], axis=1).astype(jnp.int32)

    areas = masks[0, :, -1, -1]
    areas_bc = jnp.broadcast_to(areas[:, None], (_M, _L))
    wbc = jnp.broadcast_to(
        (x2 - x0).astype(jnp.float32)[:, :, None], (_A, _G, _L))

    mask_ridx = (jnp.arange(_M, dtype=jnp.int32)[:, None, None] * _H
                 + xs[None]).reshape(_N_MASK_TASKS, 2 * _G)
    pad = _SLOTS * _NW - _N_TASKS
    ridx = jnp.concatenate(
        [mask_ridx, xs, jnp.zeros((pad, 2 * _G), jnp.int32)], axis=0)

    mesh = plsc.VectorSubcoreMesh(core_axis_name="c", subcore_axis_name="s")
    ious_flat, poke_flat = pl.kernel(
        _sc_body,
        out_type=(
            jax.ShapeDtypeStruct((_N_MASK_TASKS, _G, _G), jnp.float32),
            jax.ShapeDtypeStruct((_A, _G, _G), jnp.float32),
        ),
        mesh=mesh,
        compiler_params=pltpu.CompilerParams(needs_layout_passes=False),
        scratch_types=[
            pltpu.VMEM((_A, 2 * _G), jnp.int32),
            pltpu.VMEM((_M, _L), jnp.float32),
            pltpu.VMEM((_A, _G, _L), jnp.float32),
            pltpu.VMEM((2 * _G,), jnp.int32),
            pltpu.VMEM((2 * _G, _H), jnp.float32),
            pltpu.VMEM((_G, _G), jnp.float32),
            pltpu.SemaphoreType.DMA,
        ],
    )(masks2d, poking2d, ridx, ys, areas_bc, wbc)

    ious = ious_flat.reshape(1, _M, _A, _G, _G)
    poke = poke_flat.reshape(1, _A, _G, _G)
    return (ious, poke)


# speedup vs baseline: 2.5696x; 1.1116x over previous
import functools

import jax
import jax.numpy as jnp
from jax import lax
from jax.experimental import pallas as pl
from jax.experimental.pallas import tpu as pltpu
from jax.experimental.pallas import tpu_sc as plsc

_NC = 2
_NS = 16
_NW = _NC * _NS

_M = 16
_A = 9
_G = 32
_H = 512
_L = 16

_N_MASK_TASKS = _M * _A
_N_TASKS = _N_MASK_TASKS + _A
_SLOTS = -(-_N_TASKS // _NW)


def _sc_body(masks_hbm, poking_hbm, ridx_hbm, ys_hbm, areas_hbm, wbc_hbm,
             ious_hbm, poke_hbm,
             ys_v, areas_v, wbc_v, ridx_v, rows0_v, rows1_v, out_v,
             sem0, sem1):
    wid = lax.axis_index("s") * _NC + lax.axis_index("c")

    pltpu.sync_copy(ys_hbm, ys_v)
    pltpu.sync_copy(areas_hbm, areas_v)
    pltpu.sync_copy(wbc_hbm, wbc_v)
    pltpu.sync_copy(ridx_hbm, ridx_v)

    def start_gather(t, buf, sem):
        @pl.when(t < _N_MASK_TASKS)
        def _():
            pltpu.async_copy(masks_hbm.at[ridx_v.at[t]], buf, sem)

        @pl.when(jnp.logical_and(t >= _N_MASK_TASKS, t < _N_TASKS))
        def _():
            pltpu.async_copy(poking_hbm.at[ridx_v.at[t]], buf, sem)

    def wait_gather(t, buf, sem):
        @pl.when(t < _N_TASKS)
        def _():
            pltpu.make_async_copy(
                masks_hbm.at[pl.ds(0, 2 * _G)], buf, sem).wait()

    def compute_block(a, area_vec, use_iou, rows):
        def row_body(i, carry):
            ri = jnp.full((_L,), i, jnp.int32)
            ri2 = ri + _G
            if use_iou:
                w_vec = wbc_v[a, i, pl.ds(0, _L)]
            for jc in range(2):
                cy1 = ys_v[a, pl.ds(jc * _L, _L)]
                cy3 = ys_v[a, pl.ds(_G + jc * _L, _L)]
                v01 = plsc.load_gather(rows, [ri, cy1])
                v03 = plsc.load_gather(rows, [ri, cy3])
                v21 = plsc.load_gather(rows, [ri2, cy1])
                v23 = plsc.load_gather(rows, [ri2, cy3])
                inter = v01 + v23 - v03 - v21
                if use_iou:
                    box = w_vec * (cy3 - cy1).astype(jnp.float32)
                    res = inter / jnp.maximum(area_vec + box - inter, 1.0)
                else:
                    res = inter
                out_v[i, pl.ds(jc * _L, _L)] = res
            return carry

        lax.fori_loop(0, _G, row_body, 0)

    bufs = ((rows0_v, sem0), (rows1_v, sem1))
    start_gather(wid, rows0_v, sem0)
    for k in range(_SLOTS):
        t = wid + _NW * k
        buf, sem = bufs[k % 2]
        if k + 1 < _SLOTS:
            nbuf, nsem = bufs[(k + 1) % 2]
            start_gather(t + _NW, nbuf, nsem)
        wait_gather(t, buf, sem)

        @pl.when(t < _N_MASK_TASKS)
        def _mask_task():
            p = lax.div(t, _A)
            a = lax.rem(t, _A)
            area_vec = areas_v[p, pl.ds(0, _L)]
            compute_block(a, area_vec, True, buf)
            pltpu.sync_copy(out_v, ious_hbm.at[t])

        @pl.when(jnp.logical_and(t >= _N_MASK_TASKS, t < _N_TASKS))
        def _poke_task():
            a = t - _N_MASK_TASKS
            compute_block(a, None, False, buf)
            pltpu.sync_copy(out_v, poke_hbm.at[a])


@jax.jit
def kernel(masks, poking_locations, anchor_boxes):
    masks2d = masks.reshape(_M * _H, _H)
    poking2d = poking_locations.reshape(_H, _H)

    ab = anchor_boxes[0]
    x0 = ab[:, :, 0, 0]
    x2 = ab[:, :, 0, 2]
    y1 = ab[:, 0, :, 1]
    y3 = ab[:, 0, :, 3]
    xs = jnp.concatenate([x0, x2], axis=1).astype(jnp.int32)
    ys = jnp.concatenate([y1, y3], axis=1).astype(jnp.int32)

    areas = masks[0, :, -1, -1]
    areas_bc = jnp.broadcast_to(areas[:, None], (_M, _L))
    wbc = jnp.broadcast_to(
        (x2 - x0).astype(jnp.float32)[:, :, None], (_A, _G, _L))

    mask_ridx = (jnp.arange(_M, dtype=jnp.int32)[:, None, None] * _H
                 + xs[None]).reshape(_N_MASK_TASKS, 2 * _G)
    pad = _SLOTS * _NW - _N_TASKS
    ridx = jnp.concatenate(
        [mask_ridx, xs, jnp.zeros((pad, 2 * _G), jnp.int32)], axis=0)

    mesh = plsc.VectorSubcoreMesh(core_axis_name="c", subcore_axis_name="s")
    ious_flat, poke_flat = pl.kernel(
        _sc_body,
        out_type=(
            jax.ShapeDtypeStruct((_N_MASK_TASKS, _G, _G), jnp.float32),
            jax.ShapeDtypeStruct((_A, _G, _G), jnp.float32),
        ),
        mesh=mesh,
        compiler_params=pltpu.CompilerParams(needs_layout_passes=False),
        scratch_types=[
            pltpu.VMEM((_A, 2 * _G), jnp.int32),
            pltpu.VMEM((_M, _L), jnp.float32),
            pltpu.VMEM((_A, _G, _L), jnp.float32),
            pltpu.VMEM((_SLOTS * _NW, 2 * _G), jnp.int32),
            pltpu.VMEM((2 * _G, _H), jnp.float32),
            pltpu.VMEM((2 * _G, _H), jnp.float32),
            pltpu.VMEM((_G, _G), jnp.float32),
            pltpu.SemaphoreType.DMA,
            pltpu.SemaphoreType.DMA,
        ],
    )(masks2d, poking2d, ridx, ys, areas_bc, wbc)

    ious = ious_flat.reshape(1, _M, _A, _G, _G)
    poke = poke_flat.reshape(1, _A, _G, _G)
    return (ious, poke)


# speedup vs baseline: 2.8619x; 1.1138x over previous
import jax
import jax.numpy as jnp
from jax import lax
from jax.experimental import pallas as pl
from jax.experimental.pallas import tpu as pltpu
from jax.experimental.pallas import tpu_sc as plsc

_NC = 2
_NS = 16
_NW = _NC * _NS

_M = 16
_A = 9
_G = 32
_H = 512
_L = 16

_N_MASK_TASKS = _M * _A
_N_TASKS = _N_MASK_TASKS + _A
_SLOTS = -(-_N_TASKS // _NW)


def _sc_body(masks_hbm, poking_hbm, abf_hbm,
             ious_hbm, poke_hbm,
             ab0_v, ab1_v, arows_v, ys0_v, ys1_v, wr0_v, wr1_v,
             idx0_v, idx1_v, rows0_v, rows1_v, out_v, sem0, sem1):
    wid = lax.axis_index("s") * _NC + lax.axis_index("c")

    aidx = lax.iota(jnp.int32, 16) * _H + (_H - 1)
    pltpu.async_copy(masks_hbm.at[aidx], arows_v, sem0).wait()

    iota = lax.iota(jnp.int32, 16)

    def derive(a, off, abb, ysb, wrb, idxb):
        pltpu.sync_copy(abf_hbm.at[a], abb)
        for c in range(2):
            base = iota + c * _L
            x0 = plsc.load_gather(abb, [base * 128])
            x2 = plsc.load_gather(abb, [base * 128 + 2])
            y1 = plsc.load_gather(abb, [base * 4 + 1])
            y3 = plsc.load_gather(abb, [base * 4 + 3])
            idxb[pl.ds(c * _L, _L)] = x0 + off
            idxb[pl.ds(_G + c * _L, _L)] = x2 + off
            ysb[pl.ds(c * _L, _L)] = y1
            ysb[pl.ds(_G + c * _L, _L)] = y3
            wrb[pl.ds(c * _L, _L)] = (x2 - x0).astype(jnp.float32)

    slots = (
        (ab0_v, ys0_v, wr0_v, idx0_v, rows0_v, sem0),
        (ab1_v, ys1_v, wr1_v, idx1_v, rows1_v, sem1),
    )

    def prep_and_start(t, s):
        abb, ysb, wrb, idxb, rows, sem = slots[s]

        @pl.when(t < _N_MASK_TASKS)
        def _():
            p = lax.div(t, _A)
            a = lax.rem(t, _A)
            derive(a, p * _H, abb, ysb, wrb, idxb)
            pltpu.async_copy(masks_hbm.at[idxb], rows, sem)

        @pl.when(jnp.logical_and(t >= _N_MASK_TASKS, t < _N_TASKS))
        def _():
            a = t - _N_MASK_TASKS
            derive(a, 0, abb, ysb, wrb, idxb)
            pltpu.async_copy(poking_hbm.at[idxb], rows, sem)

    def wait_rows(t, s):
        _, _, _, _, rows, sem = slots[s]

        @pl.when(t < _N_TASKS)
        def _():
            pltpu.make_async_copy(
                masks_hbm.at[pl.ds(0, 2 * _G)], rows, sem).wait()

    def compute_block(area_vec, use_iou, s):
        _, ysb, wrb, _, rows, _ = slots[s]

        def row_body(i, carry):
            ri = jnp.full((_L,), i, jnp.int32)
            ri2 = ri + _G
            if use_iou:
                w_vec = plsc.load_gather(wrb, [ri])
            for jc in range(2):
                cy1 = ysb[pl.ds(jc * _L, _L)]
                cy3 = ysb[pl.ds(_G + jc * _L, _L)]
                v01 = plsc.load_gather(rows, [ri, cy1])
                v03 = plsc.load_gather(rows, [ri, cy3])
                v21 = plsc.load_gather(rows, [ri2, cy1])
                v23 = plsc.load_gather(rows, [ri2, cy3])
                inter = v01 + v23 - v03 - v21
                if use_iou:
                    box = w_vec * (cy3 - cy1).astype(jnp.float32)
                    res = inter / jnp.maximum(area_vec + box - inter, 1.0)
                else:
                    res = inter
                out_v[i, pl.ds(jc * _L, _L)] = res
            return carry

        lax.fori_loop(0, _G, row_body, 0)

    prep_and_start(wid, 0)
    for k in range(_SLOTS):
        t = wid + _NW * k
        s = k % 2
        if k + 1 < _SLOTS:
            prep_and_start(t + _NW, (k + 1) % 2)
        wait_rows(t, s)

        @pl.when(t < _N_MASK_TASKS)
        def _mask_task():
            p = lax.div(t, _A)
            area_vec = plsc.load_gather(
                arows_v, [jnp.full((_L,), p, jnp.int32),
                          jnp.full((_L,), _H - 1, jnp.int32)])
            compute_block(area_vec, True, s)
            pltpu.sync_copy(out_v, ious_hbm.at[t])

        @pl.when(jnp.logical_and(t >= _N_MASK_TASKS, t < _N_TASKS))
        def _poke_task():
            a = t - _N_MASK_TASKS
            compute_block(None, False, s)
            pltpu.sync_copy(out_v, poke_hbm.at[a])


@jax.jit
def kernel(masks, poking_locations, anchor_boxes):
    masks2d = masks.reshape(_M * _H, _H)
    poking2d = poking_locations.reshape(_H, _H)
    abf = anchor_boxes.reshape(_A, _G * _G * 4).astype(jnp.int32)

    mesh = plsc.VectorSubcoreMesh(core_axis_name="c", subcore_axis_name="s")
    ious_flat, poke_flat = pl.kernel(
        _sc_body,
        out_type=(
            jax.ShapeDtypeStruct((_N_MASK_TASKS, _G, _G), jnp.float32),
            jax.ShapeDtypeStruct((_A, _G, _G), jnp.float32),
        ),
        mesh=mesh,
        compiler_params=pltpu.CompilerParams(needs_layout_passes=False),
        scratch_types=[
            pltpu.VMEM((_G * _G * 4,), jnp.int32),
            pltpu.VMEM((_G * _G * 4,), jnp.int32),
            pltpu.VMEM((_M, _H), jnp.float32),
            pltpu.VMEM((2 * _G,), jnp.int32),
            pltpu.VMEM((2 * _G,), jnp.int32),
            pltpu.VMEM((_G,), jnp.float32),
            pltpu.VMEM((_G,), jnp.float32),
            pltpu.VMEM((2 * _G,), jnp.int32),
            pltpu.VMEM((2 * _G,), jnp.int32),
            pltpu.VMEM((2 * _G, _H), jnp.float32),
            pltpu.VMEM((2 * _G, _H), jnp.float32),
            pltpu.VMEM((_G, _G), jnp.float32),
            pltpu.SemaphoreType.DMA,
            pltpu.SemaphoreType.DMA,
        ],
    )(masks2d, poking2d, abf)

    ious = ious_flat.reshape(1, _M, _A, _G, _G)
    poke = poke_flat.reshape(1, _A, _G, _G)
    return (ious, poke)


# speedup vs baseline: 2.8897x; 1.0097x over previous
import jax
import jax.numpy as jnp
from jax import lax
from jax.experimental import pallas as pl
from jax.experimental.pallas import tpu as pltpu
from jax.experimental.pallas import tpu_sc as plsc

_NC = 2
_NS = 16
_NW = _NC * _NS

_M = 16
_A = 9
_G = 32
_H = 512
_L = 16

_N_MASK_TASKS = _M * _A
_N_TASKS = _N_MASK_TASKS + _A
_SLOTS = -(-_N_TASKS // _NW)


def _sc_body(masks_hbm, poking_hbm, abf_hbm,
             ious_hbm, poke_hbm,
             ab0_v, ab1_v, arows_v, ys0_v, ys1_v, wr0_v, wr1_v,
             idx0_v, idx1_v, rows0_v, rows1_v, out_v, sem0, sem1):
    wid = lax.axis_index("s") * _NC + lax.axis_index("c")

    aidx = lax.iota(jnp.int32, 16) * _H + (_H - 1)
    pltpu.async_copy(masks_hbm.at[aidx], arows_v, sem0).wait()

    iota = lax.iota(jnp.int32, 16)

    def derive(a, off, abb, ysb, wrb, idxb):
        pltpu.sync_copy(abf_hbm.at[a], abb)
        for c in range(2):
            base = iota + c * _L
            x0 = plsc.load_gather(abb, [base * 128])
            x2 = plsc.load_gather(abb, [base * 128 + 2])
            y1 = plsc.load_gather(abb, [base * 4 + 1])
            y3 = plsc.load_gather(abb, [base * 4 + 3])
            idxb[pl.ds(c * _L, _L)] = x0 + off
            idxb[pl.ds(_G + c * _L, _L)] = x2 + off
            ysb[pl.ds(c * _L, _L)] = y1
            ysb[pl.ds(_G + c * _L, _L)] = y3
            wrb[pl.ds(c * _L, _L)] = (x2 - x0).astype(jnp.float32)

    slots = (
        (ab0_v, ys0_v, wr0_v, idx0_v, rows0_v, sem0),
        (ab1_v, ys1_v, wr1_v, idx1_v, rows1_v, sem1),
    )

    def prep_and_start(t, s):
        abb, ysb, wrb, idxb, rows, sem = slots[s]

        @pl.when(t < _N_MASK_TASKS)
        def _():
            p = lax.div(t, _A)
            a = lax.rem(t, _A)
            derive(a, p * _H, abb, ysb, wrb, idxb)
            pltpu.async_copy(masks_hbm.at[idxb], rows, sem)

        @pl.when(jnp.logical_and(t >= _N_MASK_TASKS, t < _N_TASKS))
        def _():
            a = t - _N_MASK_TASKS
            derive(a, 0, abb, ysb, wrb, idxb)
            pltpu.async_copy(poking_hbm.at[idxb], rows, sem)

    def wait_rows(t, s):
        _, _, _, _, rows, sem = slots[s]

        @pl.when(t < _N_TASKS)
        def _():
            pltpu.make_async_copy(
                masks_hbm.at[pl.ds(0, 2 * _G)], rows, sem).wait()

    def compute_block(area_vec, use_iou, s):
        _, ysb, wrb, _, rows, _ = slots[s]

        @plsc.parallel_loop(0, _G, 1, unroll=2)
        def row_body(i):
            ri = jnp.full((_L,), i, jnp.int32)
            ri2 = ri + _G
            if use_iou:
                w_vec = plsc.load_gather(wrb, [ri])
            for jc in range(2):
                cy1 = ysb[pl.ds(jc * _L, _L)]
                cy3 = ysb[pl.ds(_G + jc * _L, _L)]
                v01 = plsc.load_gather(rows, [ri, cy1])
                v03 = plsc.load_gather(rows, [ri, cy3])
                v21 = plsc.load_gather(rows, [ri2, cy1])
                v23 = plsc.load_gather(rows, [ri2, cy3])
                inter = v01 + v23 - v03 - v21
                if use_iou:
                    box = w_vec * (cy3 - cy1).astype(jnp.float32)
                    res = inter / jnp.maximum(area_vec + box - inter, 1.0)
                else:
                    res = inter
                out_v[i, pl.ds(jc * _L, _L)] = res

    prep_and_start(wid, 0)
    for k in range(_SLOTS):
        t = wid + _NW * k
        s = k % 2
        if k + 1 < _SLOTS:
            prep_and_start(t + _NW, (k + 1) % 2)
        wait_rows(t, s)

        @pl.when(t < _N_MASK_TASKS)
        def _mask_task():
            p = lax.div(t, _A)
            area_vec = plsc.load_gather(
                arows_v, [jnp.full((_L,), p, jnp.int32),
                          jnp.full((_L,), _H - 1, jnp.int32)])
            compute_block(area_vec, True, s)
            pltpu.sync_copy(out_v, ious_hbm.at[t])

        @pl.when(jnp.logical_and(t >= _N_MASK_TASKS, t < _N_TASKS))
        def _poke_task():
            a = t - _N_MASK_TASKS
            compute_block(None, False, s)
            pltpu.sync_copy(out_v, poke_hbm.at[a])


@jax.jit
def kernel(masks, poking_locations, anchor_boxes):
    masks2d = masks.reshape(_M * _H, _H)
    poking2d = poking_locations.reshape(_H, _H)
    abf = anchor_boxes.reshape(_A, _G * _G * 4).astype(jnp.int32)

    mesh = plsc.VectorSubcoreMesh(core_axis_name="c", subcore_axis_name="s")
    ious_flat, poke_flat = pl.kernel(
        _sc_body,
        out_type=(
            jax.ShapeDtypeStruct((_N_MASK_TASKS, _G, _G), jnp.float32),
            jax.ShapeDtypeStruct((_A, _G, _G), jnp.float32),
        ),
        mesh=mesh,
        compiler_params=pltpu.CompilerParams(needs_layout_passes=False),
        scratch_types=[
            pltpu.VMEM((_G * _G * 4,), jnp.int32),
            pltpu.VMEM((_G * _G * 4,), jnp.int32),
            pltpu.VMEM((_M, _H), jnp.float32),
            pltpu.VMEM((2 * _G,), jnp.int32),
            pltpu.VMEM((2 * _G,), jnp.int32),
            pltpu.VMEM((_G,), jnp.float32),
            pltpu.VMEM((_G,), jnp.float32),
            pltpu.VMEM((2 * _G,), jnp.int32),
            pltpu.VMEM((2 * _G,), jnp.int32),
            pltpu.VMEM((2 * _G, _H), jnp.float32),
            pltpu.VMEM((2 * _G, _H), jnp.float32),
            pltpu.VMEM((_G, _G), jnp.float32),
            pltpu.SemaphoreType.DMA,
            pltpu.SemaphoreType.DMA,
        ],
    )(masks2d, poking2d, abf)

    ious = ious_flat.reshape(1, _M, _A, _G, _G)
    poke = poke_flat.reshape(1, _A, _G, _G)
    return (ious, poke)
